# baseline (speedup 1.0000x reference)
import numpy as np
import jax
import jax.numpy as jnp
from jax import lax
from jax.experimental import pallas as pl
from jax.experimental.pallas import tpu as pltpu
from jax.experimental.pallas import tpu_sc as plsc

_B = 4
_S = 4096
_C = 1024
_K = 512
_L = 16
_NC, _NS = 2, 16
_NW = _NC * _NS
_NG = _C // _L
_NTASK = _B * _NG
_TPW = _NTASK // _NW

_SIGN = np.int32(-2**31)


def _key_tf(kb):
    return kb ^ (lax.shift_right_arithmetic(kb, 31) & np.int32(0x7FFFFFFF))


def _digit(k, shift):
    if shift == 24:
        return lax.shift_right_logical(k ^ _SIGN, 24)
    return lax.shift_right_logical(k, shift) & np.int32(0xFF)


def _sc_body(inp, out, buf, hist, offs, outb, sem_in):
    wid = lax.axis_index("s") * _NC + lax.axis_index("c")
    lane = lax.iota(jnp.int32, _L)
    zeros = jnp.zeros((_L,), jnp.int32)
    ones = jnp.ones((_L,), jnp.int32)

    def _in_slice(task):
        b = task // _NG
        c0 = (task % _NG) * _L
        return inp.at[b, :, pl.ds(c0, _L)]

    pltpu.async_copy(_in_slice(wid * _TPW), buf.at[pl.ds(0, _S)], sem_in)

    @plsc.parallel_loop(0, 512, unroll=4)
    def _zero(i):
        hist[i, :] = zeros

    def _scan_hist(rank, dual=False):
        @plsc.parallel_loop(0, 256, unroll=4, carry=(zeros, zeros, zeros))
        def scan(i, carry):
            cum, bsel, rnew = carry
            b_ = 255 - i
            h = hist[b_, :]
            hist[b_, :] = zeros
            if dual:
                h = h + hist[b_ + 256, :]
                hist[b_ + 256, :] = zeros
            cumh = cum + h
            cond = (cum < rank) & (cumh >= rank)
            bvec = lax.broadcast(b_, (_L,)).astype(jnp.int32)
            bsel = jnp.where(cond, bvec, bsel)
            rnew = jnp.where(cond, rank - cum, rnew)
            return (cumh, bsel, rnew)
        _, bsel, rnew = scan
        return bsel, rnew

    def _task(t, carry):
        task = wid * _TPW + t
        b = task // _NG
        c0 = (task % _NG) * _L
        pltpu.make_async_copy(_in_slice(task), buf.at[pl.ds(0, _S)],
                              sem_in).wait()

        @plsc.parallel_loop(0, _S, unroll=8)
        def p1(r):
            kb = plsc.bitcast(buf[r, :], jnp.int32)
            plsc.addupdate_scatter(
                hist, [_digit(_key_tf(kb), 24) + (r & 1) * 256, lane], ones)
        b1, rank = _scan_hist(jnp.full((_L,), _K, jnp.int32), dual=True)

        def classify(n_rows, bsel, shift, co, cc_bound, raw):
            def body(r, carry):
                co_, cc_ = carry
                kf = buf[r, :]
                k = plsc.bitcast(kf, jnp.int32)
                if raw:
                    k = _key_tf(k)
                    kf = plsc.bitcast(k, jnp.float32)
                d = _digit(k, shift)
                if cc_bound is None:
                    m_gt = d > bsel
                    m_eq = d == bsel
                else:
                    valid = lax.broadcast(r, (_L,)) < cc_bound
                    m_gt = valid & (d > bsel)
                    m_eq = valid & (d == bsel)
                idx = jnp.where(m_gt, co_ + _S, cc_)
                plsc.store_scatter(buf, [idx, lane], kf, mask=m_gt | m_eq)
                co_ = co_ + jnp.where(m_gt, ones, zeros)
                cc_ = cc_ + jnp.where(m_eq, ones, zeros)
                return (co_, cc_)
            return plsc.parallel_loop(0, n_rows,
                                      unroll=8 if cc_bound is None else 4,
                                      carry=(co, zeros))(body)

        def cand_hist(cc, shift):
            @plsc.parallel_loop(0, jnp.max(cc), unroll=4)
            def h(r):
                k = plsc.bitcast(buf[r, :], jnp.int32)
                plsc.addupdate_scatter(hist, [_digit(k, shift), lane], ones,
                                       mask=lax.broadcast(r, (_L,)) < cc)

        co, cc = classify(_S, b1, 24, zeros, None, True)
        cand_hist(cc, 16)
        b2, rank = _scan_hist(rank)
        co, cc = classify(jnp.max(cc), b2, 16, co, cc, False)
        cand_hist(cc, 8)
        b3, rank = _scan_hist(rank)
        co, cc = classify(jnp.max(cc), b3, 8, co, cc, False)
        cand_hist(cc, 0)
        b4, rank = _scan_hist(rank)
        co, cc = classify(jnp.max(cc), b4, 0, co, cc, False)

        tkey = ((b1 * 16777216) + (b2 * 65536) + (b3 * 256) + b4) ^ _SIGN
        tf = plsc.bitcast(tkey, jnp.float32)

        @plsc.parallel_loop(jnp.min(co), _K)
        def fill(r):
            rv = lax.broadcast(r, (_L,))
            plsc.store_scatter(buf, [rv + _S, lane], tf, mask=rv >= co)

        nxt = jnp.minimum(t + 1, _TPW - 1) + wid * _TPW
        pltpu.async_copy(_in_slice(nxt), buf.at[pl.ds(0, _S)], sem_in)

        def sort_pass(src, dst, shift, last):
            @plsc.parallel_loop(0, _K, unroll=8)
            def h_(r):
                if src == 0:
                    k = plsc.bitcast(buf[r + _S, :], jnp.int32)
                else:
                    k = plsc.bitcast(outb[r, :], jnp.int32)
                plsc.addupdate_scatter(
                    hist, [_digit(k, shift) + (r & 1) * 256, lane], ones)

            @plsc.parallel_loop(0, 256, unroll=4, carry=zeros)
            def o_(i, cum):
                b_ = 255 - i
                h = hist[b_, :] + hist[b_ + 256, :]
                hist[b_, :] = zeros
                hist[b_ + 256, :] = zeros
                offs[b_, :] = cum
                return cum + h

            def r_(g, c):
                r0 = g * 8
                kf, k, d, pos = [], [], [], []
                for i in range(8):
                    if src == 0:
                        kfi = buf[r0 + i + _S, :]
                    else:
                        kfi = outb[r0 + i, :]
                    ki = plsc.bitcast(kfi, jnp.int32)
                    kf.append(kfi)
                    k.append(ki)
                    d.append(_digit(ki, shift))
                    pos.append(plsc.load_gather(offs, [d[i], lane]))
                for i in range(1, 8):
                    adj = zeros
                    for j in range(i):
                        adj = adj + jnp.where(d[i] == d[j], ones, zeros)
                    pos[i] = pos[i] + adj
                for i in range(8):
                    val = (plsc.bitcast(_key_tf(k[i]), jnp.float32)
                           if last else kf[i])
                    if dst == 0:
                        plsc.store_scatter(buf, [pos[i] + _S, lane], val)
                    else:
                        plsc.store_scatter(outb, [pos[i], lane], val)
                for i in range(8):
                    plsc.store_scatter(offs, [d[i], lane], pos[i] + ones)
                return c
            lax.fori_loop(0, _K // 8, r_, 0)

        sort_pass(0, 1, 0, False)
        sort_pass(1, 0, 8, False)
        sort_pass(0, 1, 16, False)
        sort_pass(1, 0, 24, True)

        pltpu.sync_copy(buf.at[pl.ds(_S, _K)], out.at[b, :, pl.ds(c0, _L)])
        return carry

    lax.fori_loop(0, _TPW, _task, 0)
    pltpu.make_async_copy(_in_slice(wid * _TPW + _TPW - 1),
                          buf.at[pl.ds(0, _S)], sem_in).wait()


def kernel(inputs):
    mesh = plsc.VectorSubcoreMesh(core_axis_name="c", subcore_axis_name="s",
                                  num_cores=_NC, num_subcores=_NS)
    f = pl.kernel(
        _sc_body,
        out_type=jax.ShapeDtypeStruct((_B, _K, _C), jnp.float32),
        mesh=mesh,
        compiler_params=pltpu.CompilerParams(use_tc_tiling_on_sc=False,
                                             needs_layout_passes=False),
        scratch_types=[
            pltpu.VMEM((_S + _K, _L), jnp.float32),
            pltpu.VMEM((512, _L), jnp.int32),
            pltpu.VMEM((256, _L), jnp.int32),
            pltpu.VMEM((_K, _L), jnp.float32),
            pltpu.SemaphoreType.DMA,
        ],
    )
    return f(inputs)


# speedup vs baseline: 1.0128x; 1.0128x over previous
import numpy as np
import jax
import jax.numpy as jnp
from jax import lax
from jax.experimental import pallas as pl
from jax.experimental.pallas import tpu as pltpu
from jax.experimental.pallas import tpu_sc as plsc

_B = 4
_S = 4096
_C = 1024
_K = 512
_L = 16
_NC, _NS = 2, 16
_NW = _NC * _NS
_NG = _C // _L
_NTASK = _B * _NG
_TPW = _NTASK // _NW

_SIGN = np.int32(-2**31)


def _key_tf(kb):
    return kb ^ (lax.shift_right_arithmetic(kb, 31) & np.int32(0x7FFFFFFF))


def _digit(k, shift):
    if shift == 24:
        return lax.shift_right_logical(k ^ _SIGN, 24)
    return lax.shift_right_logical(k, shift) & np.int32(0xFF)


def _sc_body(inp, out, buf, hist, offs, outb, sem_in):
    wid = lax.axis_index("s") * _NC + lax.axis_index("c")
    lane = lax.iota(jnp.int32, _L)
    zeros = jnp.zeros((_L,), jnp.int32)
    ones = jnp.ones((_L,), jnp.int32)

    def _in_slice(task):
        b = task // _NG
        c0 = (task % _NG) * _L
        return inp.at[b, :, pl.ds(c0, _L)]

    pltpu.async_copy(_in_slice(wid * _TPW), buf.at[pl.ds(0, _S)], sem_in)

    @plsc.parallel_loop(0, 512, unroll=4)
    def _zero(i):
        hist[i, :] = zeros

    def _scan_hist(rank, dual=False):
        @plsc.parallel_loop(0, 256, unroll=4, carry=(zeros, zeros, zeros))
        def scan(i, carry):
            cum, bsel, rnew = carry
            b_ = 255 - i
            h = hist[b_, :]
            hist[b_, :] = zeros
            if dual:
                h = h + hist[b_ + 256, :]
                hist[b_ + 256, :] = zeros
            cumh = cum + h
            cond = (cum < rank) & (cumh >= rank)
            bvec = lax.broadcast(b_, (_L,)).astype(jnp.int32)
            bsel = jnp.where(cond, bvec, bsel)
            rnew = jnp.where(cond, rank - cum, rnew)
            return (cumh, bsel, rnew)
        _, bsel, rnew = scan
        return bsel, rnew

    def _task(t, carry):
        task = wid * _TPW + t
        b = task // _NG
        c0 = (task % _NG) * _L
        pltpu.make_async_copy(_in_slice(task), buf.at[pl.ds(0, _S)],
                              sem_in).wait()

        @plsc.parallel_loop(0, _S, unroll=8)
        def p1(r):
            kb = plsc.bitcast(buf[r, :], jnp.int32)
            plsc.addupdate_scatter(
                hist, [_digit(_key_tf(kb), 24) + (r & 1) * 256, lane], ones)
        b1, rank = _scan_hist(jnp.full((_L,), _K, jnp.int32), dual=True)

        def classify(n_rows, bsel, shift, co, cc_bound, raw):
            def body(r, carry):
                co_, cc_ = carry
                kf = buf[r, :]
                k = plsc.bitcast(kf, jnp.int32)
                if raw:
                    k = _key_tf(k)
                    kf = plsc.bitcast(k, jnp.float32)
                d = _digit(k, shift)
                if cc_bound is None:
                    m_gt = d > bsel
                    m_eq = d == bsel
                else:
                    valid = lax.broadcast(r, (_L,)) < cc_bound
                    m_gt = valid & (d > bsel)
                    m_eq = valid & (d == bsel)
                idx = jnp.where(m_gt, co_ + _S, cc_)
                plsc.store_scatter(buf, [idx, lane], kf, mask=m_gt | m_eq)
                co_ = co_ + jnp.where(m_gt, ones, zeros)
                cc_ = cc_ + jnp.where(m_eq, ones, zeros)
                return (co_, cc_)
            return plsc.parallel_loop(0, n_rows,
                                      unroll=8 if cc_bound is None else 4,
                                      carry=(co, zeros))(body)

        def cand_hist(cc, shift):
            @plsc.parallel_loop(0, jnp.max(cc), unroll=4)
            def h(r):
                k = plsc.bitcast(buf[r, :], jnp.int32)
                plsc.addupdate_scatter(hist, [_digit(k, shift), lane], ones,
                                       mask=lax.broadcast(r, (_L,)) < cc)

        co, cc = classify(_S, b1, 24, zeros, None, True)
        cand_hist(cc, 16)
        b2, rank = _scan_hist(rank)
        co, cc = classify(jnp.max(cc), b2, 16, co, cc, False)
        cand_hist(cc, 8)
        b3, rank = _scan_hist(rank)
        co, cc = classify(jnp.max(cc), b3, 8, co, cc, False)
        cand_hist(cc, 0)
        b4, rank = _scan_hist(rank)
        co, cc = classify(jnp.max(cc), b4, 0, co, cc, False)

        tkey = ((b1 * 16777216) + (b2 * 65536) + (b3 * 256) + b4) ^ _SIGN
        tf = plsc.bitcast(tkey, jnp.float32)

        @plsc.parallel_loop(jnp.min(co), _K)
        def fill(r):
            rv = lax.broadcast(r, (_L,))
            plsc.store_scatter(buf, [rv + _S, lane], tf, mask=rv >= co)

        nxt = jnp.minimum(t + 1, _TPW - 1) + wid * _TPW
        pltpu.async_copy(_in_slice(nxt), buf.at[pl.ds(0, _S)], sem_in)

        def sort_pass(src, dst, shift, last):
            @plsc.parallel_loop(0, _K, unroll=8)
            def h_(r):
                if src == 0:
                    k = plsc.bitcast(buf[r + _S, :], jnp.int32)
                else:
                    k = plsc.bitcast(outb[r, :], jnp.int32)
                plsc.addupdate_scatter(hist, [_digit(k, shift), lane], ones)

            @plsc.parallel_loop(0, 256, unroll=4, carry=zeros)
            def o_(i, cum):
                b_ = 255 - i
                h = hist[b_, :]
                hist[b_, :] = zeros
                offs[b_, :] = cum
                return cum + h

            def r_(g, c):
                r0 = g * 8
                kf, k, d, pos = [], [], [], []
                for i in range(8):
                    if src == 0:
                        kfi = buf[r0 + i + _S, :]
                    else:
                        kfi = outb[r0 + i, :]
                    ki = plsc.bitcast(kfi, jnp.int32)
                    kf.append(kfi)
                    k.append(ki)
                    d.append(_digit(ki, shift))
                    pos.append(plsc.load_gather(offs, [d[i], lane]))
                for i in range(1, 8):
                    adj = zeros
                    for j in range(i):
                        adj = adj + jnp.where(d[i] == d[j], ones, zeros)
                    pos[i] = pos[i] + adj
                for i in range(8):
                    val = (plsc.bitcast(_key_tf(k[i]), jnp.float32)
                           if last else kf[i])
                    if dst == 0:
                        plsc.store_scatter(buf, [pos[i] + _S, lane], val)
                    else:
                        plsc.store_scatter(outb, [pos[i], lane], val)
                for i in range(8):
                    plsc.store_scatter(offs, [d[i], lane], pos[i] + ones)
                return c
            lax.fori_loop(0, _K // 8, r_, 0)

        sort_pass(0, 1, 0, False)
        sort_pass(1, 0, 8, False)
        sort_pass(0, 1, 16, False)
        sort_pass(1, 0, 24, True)

        pltpu.sync_copy(buf.at[pl.ds(_S, _K)], out.at[b, :, pl.ds(c0, _L)])
        return carry

    lax.fori_loop(0, _TPW, _task, 0)
    pltpu.make_async_copy(_in_slice(wid * _TPW + _TPW - 1),
                          buf.at[pl.ds(0, _S)], sem_in).wait()


def kernel(inputs):
    mesh = plsc.VectorSubcoreMesh(core_axis_name="c", subcore_axis_name="s",
                                  num_cores=_NC, num_subcores=_NS)
    f = pl.kernel(
        _sc_body,
        out_type=jax.ShapeDtypeStruct((_B, _K, _C), jnp.float32),
        mesh=mesh,
        compiler_params=pltpu.CompilerParams(use_tc_tiling_on_sc=False,
                                             needs_layout_passes=False),
        scratch_types=[
            pltpu.VMEM((_S + _K, _L), jnp.float32),
            pltpu.VMEM((512, _L), jnp.int32),
            pltpu.VMEM((256, _L), jnp.int32),
            pltpu.VMEM((_K, _L), jnp.float32),
            pltpu.SemaphoreType.DMA,
        ],
    )
    return f(inputs)
